# Initial kernel scaffold; baseline (speedup 1.0000x reference)
import jax
import jax.numpy as jnp
from jax.experimental import pallas as pl


def kernel(x, edge_index, edge_type, batch, shape_emb, color_emb, pos_emb, W1, root1, b1, g1, be1, W2, root2, b2, g2, be2, clsW, clsb):
    raise NotImplementedError("write your pallas kernel here")


# speedup vs baseline: 1.2111x; 1.2111x over previous
import functools

import jax
import jax.numpy as jnp
from jax import lax
from jax.experimental import pallas as pl
from jax.experimental.pallas import tpu as pltpu
from jax.experimental.pallas import tpu_sc as plsc

N = 50000
E = 800000
G = 2000
NREL = 3
ED = 32
H = 64
NCLS = 10

NC = 2
NS = 16
NH = N // NC
RACC = NREL * NH + 8
TRASH = NREL * NH
DRAIN = RACC // NS

NB = 16
OUT_IT = 25
ROWS_PW = OUT_IT * NB
ER_TOT = ROWS_PW * NS
EPAD = ER_TOT * 128

NPOOL = 53248
PR_TOT = NPOOL // 128
PR_PW = PR_TOT // (NC * NS)
GACC = 2048
TRASHG = 2040

RB = 200
NGRID = N // RB


def _make_edge_agg(tab_rows):
  mesh = plsc.VectorSubcoreMesh(core_axis_name="c", subcore_axis_name="s")

  @functools.partial(
      pl.kernel,
      out_type=jax.ShapeDtypeStruct((NC, RACC, 16), jnp.float32),
      mesh=mesh,
      compiler_params=pltpu.CompilerParams(use_tc_tiling_on_sc=False),
      scratch_types=[
          pltpu.VMEM((NB, 128), jnp.int32),
          pltpu.VMEM((NB, 128), jnp.int32),
          pltpu.VMEM((NB, 128, 16), jnp.float32),
          pltpu.VMEM((128, 16), jnp.float32),
          pltpu.VMEM_SHARED((RACC, 16), jnp.float32),
          pltpu.SemaphoreType.DMA,
          pltpu.SemaphoreType.DMA,
      ],
  )
  def agg(tab_hbm, src_hbm, sidx_hbm, out_hbm,
          gi_v, si_v, rows_v, zb_v, acc_sh, gsem, ssem):
    c = lax.axis_index("c")
    s = lax.axis_index("s")

    def zero_body(i, carry):
      zb_v[i] = jnp.zeros((16,), jnp.float32)
      return carry

    lax.fori_loop(0, 128, zero_body, 0)

    def zero_copy(t, carry):
      pltpu.sync_copy(zb_v, acc_sh.at[pl.ds(s * DRAIN + t * 128, 128)])
      return carry

    lax.fori_loop(0, DRAIN // 128, zero_copy, 0)
    rem = DRAIN % 128
    if rem:
      pltpu.sync_copy(zb_v.at[pl.ds(0, rem)],
                      acc_sh.at[pl.ds(s * DRAIN + DRAIN - rem, rem)])
    plsc.subcore_barrier()

    def outer(i, carry):
      rb = s * ROWS_PW + i * NB
      pltpu.sync_copy(src_hbm.at[pl.ds(rb, NB)], gi_v)
      pltpu.sync_copy(sidx_hbm.at[c].at[pl.ds(rb, NB)], si_v)
      gds = [pltpu.async_copy(tab_hbm.at[gi_v.at[j]], rows_v.at[j], gsem)
             for j in range(NB)]
      for d in gds:
        d.wait()
      sds = [pltpu.async_copy(rows_v.at[j], acc_sh.at[si_v.at[j]], ssem,
                              add=True)
             for j in range(NB)]
      for d in sds:
        d.wait()
      return carry

    lax.fori_loop(0, OUT_IT, outer, 0)
    plsc.subcore_barrier()
    pltpu.sync_copy(acc_sh.at[pl.ds(s * DRAIN, DRAIN)],
                    out_hbm.at[c].at[pl.ds(s * DRAIN, DRAIN)])

  return agg


_edge_agg_tab = _make_edge_agg(N)
_edge_agg_cnt = _make_edge_agg(8)


def _make_pool():
  mesh = plsc.VectorSubcoreMesh(core_axis_name="c", subcore_axis_name="s")

  @functools.partial(
      pl.kernel,
      out_type=(jax.ShapeDtypeStruct((NC, GACC, H), jnp.float32),
                jax.ShapeDtypeStruct((NC, GACC, 16), jnp.float32)),
      mesh=mesh,
      compiler_params=pltpu.CompilerParams(use_tc_tiling_on_sc=False),
      scratch_types=[
          pltpu.VMEM((PR_PW, 128), jnp.int32),
          pltpu.VMEM((7, 128, H), jnp.float32),
          pltpu.VMEM((128, H), jnp.float32),
          pltpu.VMEM((128, 16), jnp.float32),
          pltpu.VMEM_SHARED((GACC, H), jnp.float32),
          pltpu.VMEM_SHARED((GACC, 16), jnp.float32),
          pltpu.SemaphoreType.DMA,
          pltpu.SemaphoreType.DMA,
      ],
  )
  def pool(h2_hbm, bidx_hbm, pout_hbm, cout_hbm,
           bi_v, rows_v, zb_v, ob_v, pacc_sh, cacc_sh, psem, csem):
    c = lax.axis_index("c")
    s = lax.axis_index("s")
    w = s * NC + c

    def zero_wide(i, carry):
      for q in range(H // 16):
        zb_v[i, q * 16:(q + 1) * 16] = jnp.zeros((16,), jnp.float32)
      return carry

    lax.fori_loop(0, 128, zero_wide, 0)

    def zero_narrow(i, carry):
      ob_v[i] = jnp.zeros((16,), jnp.float32)
      return carry

    lax.fori_loop(0, 128, zero_narrow, 0)
    pltpu.sync_copy(zb_v, pacc_sh.at[pl.ds(s * 128, 128)])
    pltpu.sync_copy(ob_v, cacc_sh.at[pl.ds(s * 128, 128)])

    def fill_ones(i, carry):
      ob_v[i] = jnp.ones((16,), jnp.float32)
      return carry

    lax.fori_loop(0, 128, fill_ones, 0)
    plsc.subcore_barrier()

    pltpu.sync_copy(bidx_hbm.at[pl.ds(w * PR_PW, PR_PW)], bi_v)
    for off, nb in ((0, 7), (7, 6)):
      pltpu.sync_copy(h2_hbm.at[pl.ds(w * PR_PW + off, nb)],
                      rows_v.at[pl.ds(0, nb)])
      pds = [pltpu.async_copy(rows_v.at[j], pacc_sh.at[bi_v.at[off + j]],
                              psem, add=True)
             for j in range(nb)]
      cds = [pltpu.async_copy(ob_v, cacc_sh.at[bi_v.at[off + j]], csem,
                              add=True)
             for j in range(nb)]
      for d in pds:
        d.wait()
      for d in cds:
        d.wait()
    plsc.subcore_barrier()
    pltpu.sync_copy(pacc_sh.at[pl.ds(s * 128, 128)],
                    pout_hbm.at[c].at[pl.ds(s * 128, 128)])
    pltpu.sync_copy(cacc_sh.at[pl.ds(s * 128, 128)],
                    cout_hbm.at[c].at[pl.ds(s * 128, 128)])

  return pool


_pool = _make_pool()


def _embed_body(x_ref, se_ref, ce_ref, pe_ref, o0_ref, o1_ref):
  xb = x_ref[...]
  oh0 = (xb[:, 0:1] ==
         lax.broadcasted_iota(jnp.int32, (RB, 8), 1)).astype(jnp.float32)
  oh1 = (xb[:, 1:2] ==
         lax.broadcasted_iota(jnp.int32, (RB, 8), 1)).astype(jnp.float32)
  p = jnp.clip(xb[:, 2:3], 0, 24)
  ohp = (p == lax.broadcasted_iota(jnp.int32, (RB, 32), 1)).astype(jnp.float32)
  h = (jnp.dot(oh0, se_ref[...], preferred_element_type=jnp.float32)
       + jnp.dot(oh1, ce_ref[...], preferred_element_type=jnp.float32)
       + jnp.dot(ohp, pe_ref[...], preferred_element_type=jnp.float32))
  o0_ref[...] = h[:, :16]
  o1_ref[...] = h[:, 16:]


def _embed(x, se, ce, pe_pad):
  full = lambda shp: pl.BlockSpec(shp, lambda n: tuple(0 for _ in shp))
  return pl.pallas_call(
      _embed_body,
      grid=(NGRID,),
      in_specs=[
          pl.BlockSpec((RB, 3), lambda n: (n, 0)),
          full((8, ED)), full((8, ED)), full((32, ED)),
      ],
      out_specs=[pl.BlockSpec((RB, 16), lambda n: (n, 0)),
                 pl.BlockSpec((RB, 16), lambda n: (n, 0))],
      out_shape=[jax.ShapeDtypeStruct((N, 16), jnp.float32),
                 jax.ShapeDtypeStruct((N, 16), jnp.float32)],
  )(x, se, ce, pe_pad)


def _dense_layer(h_chunks, s_raw_chunks, cnt_raw, W, root):
  C = len(h_chunks)
  din = 16 * C

  def body(*refs):
    h_refs = refs[:C]
    s_refs = refs[C:C + NREL * C]
    cnt_refs = refs[C + NREL * C:C + NREL * C + NREL]
    w_ref, root_ref = refs[C + NREL * C + NREL:C + NREL * C + NREL + 2]
    z_ref, st_ref = refs[-2:]

    n = pl.program_id(0)
    hb = jnp.concatenate([r[...] for r in h_refs], axis=1)
    z = jnp.dot(hb, root_ref[...], preferred_element_type=jnp.float32)
    for r in range(NREL):
      acc = jnp.zeros((RB, H), jnp.float32)
      for k in range(C):
        sk = s_refs[r * C + k][0]
        acc = acc + jnp.dot(sk, w_ref[r, 16 * k:16 * (k + 1), :],
                            preferred_element_type=jnp.float32)
      cntr = cnt_refs[r][0][:, 0]
      z = z + acc / jnp.maximum(cntr, 1.0)[:, None]
    z_ref[...] = z

    @pl.when(n == 0)
    def _():
      st_ref[...] = jnp.zeros((8, H), jnp.float32)

    st_ref[0:1, :] += jnp.sum(z, axis=0)[None]
    st_ref[1:2, :] += jnp.sum(z * z, axis=0)[None]

  def smap(r):
    return pl.BlockSpec(
        (1, RB, 16),
        lambda n, r=r: (n // (NH // RB), r * (NH // RB) + n % (NH // RB), 0))

  in_specs = ([pl.BlockSpec((RB, 16), lambda n: (n, 0))] * C
              + [smap(r) for r in range(NREL) for _ in range(C)]
              + [smap(r) for r in range(NREL)]
              + [pl.BlockSpec((NREL, din, H), lambda n: (0, 0, 0)),
                 pl.BlockSpec((din, H), lambda n: (0, 0))])
  operands = (list(h_chunks)
              + [s_raw_chunks[k] for _ in range(NREL) for k in range(C)]
              + [cnt_raw] * NREL + [W, root])
  return pl.pallas_call(
      body,
      grid=(NGRID,),
      in_specs=in_specs,
      out_specs=[pl.BlockSpec((RB, H), lambda n: (n, 0)),
                 pl.BlockSpec((8, H), lambda n: (0, 0))],
      out_shape=[jax.ShapeDtypeStruct((N, H), jnp.float32),
                 jax.ShapeDtypeStruct((8, H), jnp.float32)],
  )(*operands)


def _bn_relu(z, stats, gb, n_chunks):
  def body(z_ref, st_ref, gb_ref, *o_refs):
    zb = z_ref[...]
    mu = st_ref[0, :] / N
    var = st_ref[1, :] / N - mu * mu
    inv = lax.rsqrt(var + 1e-5)
    h = jnp.maximum(gb_ref[0, :] * (zb - mu) * inv + gb_ref[1, :], 0.0)
    if n_chunks == 0:
      o_refs[0][...] = h
    else:
      for k in range(n_chunks):
        o_refs[k][...] = h[:, 16 * k:16 * (k + 1)]

  if n_chunks == 0:
    out_specs = [pl.BlockSpec((RB, H), lambda n: (n, 0))]
    out_shape = [jax.ShapeDtypeStruct((N, H), jnp.float32)]
  else:
    out_specs = [pl.BlockSpec((RB, 16), lambda n: (n, 0))] * n_chunks
    out_shape = [jax.ShapeDtypeStruct((N, 16), jnp.float32)] * n_chunks
  res = pl.pallas_call(
      body,
      grid=(NGRID,),
      in_specs=[pl.BlockSpec((RB, H), lambda n: (n, 0)),
                pl.BlockSpec((8, H), lambda n: (0, 0)),
                pl.BlockSpec((8, H), lambda n: (0, 0))],
      out_specs=out_specs,
      out_shape=out_shape,
  )(z, stats, gb)
  return res


def _classifier(p0, p1, c0, c1, wpad, cbpad):
  def body(p0_ref, p1_ref, c0_ref, c1_ref, w_ref, cb_ref, o_ref):
    p = p0_ref[...] + p1_ref[...]
    cnt = c0_ref[...][:, 0:1] + c1_ref[...][:, 0:1]
    hg = p / jnp.maximum(cnt, 1.0)
    o_ref[...] = (jnp.dot(hg, w_ref[...], preferred_element_type=jnp.float32)
                  + cb_ref[0:1, :])

  return pl.pallas_call(
      body,
      out_shape=jax.ShapeDtypeStruct((GACC, 16), jnp.float32),
  )(p0, p1, c0, c1, wpad, cbpad)


def kernel(x, edge_index, edge_type, batch, shape_emb, color_emb, pos_emb,
           W1, root1, b1, g1, be1, W2, root2, b2, g2, be2, clsW, clsb):
  x = x.astype(jnp.int32)
  src = edge_index[0].astype(jnp.int32)
  dst = edge_index[1].astype(jnp.int32)
  et = edge_type.astype(jnp.int32)

  pad = EPAD - E
  src_p = jnp.concatenate([src, jnp.zeros((pad,), jnp.int32)])
  src_p = src_p.reshape(ER_TOT, 128)
  halves = []
  for c in range(NC):
    loc = et * NH + dst - c * NH
    ok = (dst >= c * NH) & (dst < (c + 1) * NH)
    hi = jnp.where(ok, loc, TRASH)
    halves.append(jnp.concatenate([hi, jnp.full((pad,), TRASH, jnp.int32)]))
  sidx2 = jnp.stack(halves).reshape(NC, ER_TOT, 128)
  zidx = jnp.zeros((ER_TOT, 128), jnp.int32)
  ones_tab = jnp.ones((8, 16), jnp.float32)

  pe_pad = jnp.zeros((32, ED), jnp.float32).at[:25].set(pos_emb)
  h0c = _embed(x, shape_emb, color_emb, pe_pad)

  cnt_raw = _edge_agg_cnt(ones_tab, zidx, sidx2)

  s1 = [_edge_agg_tab(h0c[k], src_p, sidx2) for k in range(ED // 16)]
  z1, st1 = _dense_layer(h0c, s1, cnt_raw, W1, root1)
  gb1 = jnp.zeros((8, H), jnp.float32).at[0].set(g1).at[1].set(be1)
  h1c = _bn_relu(z1, st1, gb1, H // 16)

  s2 = [_edge_agg_tab(h1c[k], src_p, sidx2) for k in range(H // 16)]
  z2, st2 = _dense_layer(h1c, s2, cnt_raw, W2, root2)
  gb2 = jnp.zeros((8, H), jnp.float32).at[0].set(g2).at[1].set(be2)
  h2 = _bn_relu(z2, st2, gb2, 0)[0]

  h2p = jnp.concatenate(
      [h2, jnp.zeros((NPOOL - N, H), jnp.float32)]).reshape(PR_TOT, 128, H)
  bidx = jnp.concatenate(
      [batch.astype(jnp.int32),
       jnp.full((NPOOL - N,), TRASHG, jnp.int32)]).reshape(PR_TOT, 128)
  pout, cout = _pool(h2p, bidx)

  wpad = jnp.zeros((H, 16), jnp.float32).at[:, :NCLS].set(clsW)
  cbpad = jnp.zeros((8, 16), jnp.float32).at[0, :NCLS].set(clsb)
  logits = _classifier(pout[0], pout[1], cout[0], cout[1], wpad, cbpad)
  return logits[:G, :NCLS]


# speedup vs baseline: 2.8772x; 2.3757x over previous
import functools

import jax
import jax.numpy as jnp
from jax import lax
from jax.experimental import pallas as pl
from jax.experimental.pallas import tpu as pltpu
from jax.experimental.pallas import tpu_sc as plsc

N = 50000
E = 800000
G = 2000
NREL = 3
ED = 32
H = 64
NCLS = 10

NC = 2
NS = 16
NH = N // NC
RACC = NREL * NH + 8
TRASH = NREL * NH
DRAIN = RACC // NS

NB = 16
OUT_IT = 25
ROWS_PW = OUT_IT * NB
ER_TOT = ROWS_PW * NS
EPAD = ER_TOT * 128

NPOOL = 53248
PR_TOT = NPOOL // 128
PR_PW = PR_TOT // (NC * NS)
GACC = 2048
TRASHG = 2040

RB = 200
NGRID = N // RB


def _make_edge_agg(tab_rows, gather=True):
  mesh = plsc.VectorSubcoreMesh(core_axis_name="c", subcore_axis_name="s")

  @functools.partial(
      pl.kernel,
      out_type=jax.ShapeDtypeStruct((NC, RACC, 16), jnp.float32),
      mesh=mesh,
      compiler_params=pltpu.CompilerParams(use_tc_tiling_on_sc=False),
      scratch_types=[
          pltpu.VMEM((NB, 128), jnp.int32),
          pltpu.VMEM((NB, 128), jnp.int32),
          pltpu.VMEM((NB, 128, 16), jnp.float32),
          pltpu.VMEM((128, 16), jnp.float32),
          pltpu.VMEM_SHARED((RACC, 16), jnp.float32),
          pltpu.SemaphoreType.DMA,
          pltpu.SemaphoreType.DMA,
      ],
  )
  def agg(tab_hbm, src_hbm, sidx_hbm, out_hbm,
          gi_v, si_v, rows_v, zb_v, acc_sh, gsem, ssem):
    c = lax.axis_index("c")
    s = lax.axis_index("s")

    def zero_body(i, carry):
      zb_v[i] = jnp.zeros((16,), jnp.float32)
      return carry

    lax.fori_loop(0, 128, zero_body, 0)

    def zero_copy(t, carry):
      pltpu.sync_copy(zb_v, acc_sh.at[pl.ds(s * DRAIN + t * 128, 128)])
      return carry

    lax.fori_loop(0, DRAIN // 128, zero_copy, 0)
    rem = DRAIN % 128
    if rem:
      pltpu.sync_copy(zb_v.at[pl.ds(0, rem)],
                      acc_sh.at[pl.ds(s * DRAIN + DRAIN - rem, rem)])
    if not gather:
      def ones_body(i, carry):
        for j in range(NB):
          rows_v[j, i] = jnp.ones((16,), jnp.float32)
        return carry

      lax.fori_loop(0, 128, ones_body, 0)
    plsc.subcore_barrier()

    def outer(i, carry):
      rb = s * ROWS_PW + i * NB
      pltpu.sync_copy(sidx_hbm.at[c].at[pl.ds(rb, NB)], si_v)
      if gather:
        pltpu.sync_copy(src_hbm.at[pl.ds(rb, NB)], gi_v)
        gds = [pltpu.async_copy(tab_hbm.at[gi_v.at[j]], rows_v.at[j], gsem)
               for j in range(NB)]
        for d in gds:
          d.wait()
      sds = [pltpu.async_copy(rows_v.at[j], acc_sh.at[si_v.at[j]], ssem,
                              add=True)
             for j in range(NB)]
      for d in sds:
        d.wait()
      return carry

    lax.fori_loop(0, OUT_IT, outer, 0)
    plsc.subcore_barrier()
    pltpu.sync_copy(acc_sh.at[pl.ds(s * DRAIN, DRAIN)],
                    out_hbm.at[c].at[pl.ds(s * DRAIN, DRAIN)])

  return agg


_edge_agg_tab = _make_edge_agg(N)
_edge_agg_cnt = _make_edge_agg(8, gather=False)


def _make_pool():
  mesh = plsc.VectorSubcoreMesh(core_axis_name="c", subcore_axis_name="s")

  @functools.partial(
      pl.kernel,
      out_type=(jax.ShapeDtypeStruct((NC, GACC, H), jnp.float32),
                jax.ShapeDtypeStruct((NC, GACC, 16), jnp.float32)),
      mesh=mesh,
      compiler_params=pltpu.CompilerParams(use_tc_tiling_on_sc=False),
      scratch_types=[
          pltpu.VMEM((PR_PW, 128), jnp.int32),
          pltpu.VMEM((7, 128, H), jnp.float32),
          pltpu.VMEM((128, H), jnp.float32),
          pltpu.VMEM((128, 16), jnp.float32),
          pltpu.VMEM_SHARED((GACC, H), jnp.float32),
          pltpu.VMEM_SHARED((GACC, 16), jnp.float32),
          pltpu.SemaphoreType.DMA,
          pltpu.SemaphoreType.DMA,
      ],
  )
  def pool(h2_hbm, bidx_hbm, pout_hbm, cout_hbm,
           bi_v, rows_v, zb_v, ob_v, pacc_sh, cacc_sh, psem, csem):
    c = lax.axis_index("c")
    s = lax.axis_index("s")
    w = s * NC + c

    def zero_wide(i, carry):
      for q in range(H // 16):
        zb_v[i, q * 16:(q + 1) * 16] = jnp.zeros((16,), jnp.float32)
      return carry

    lax.fori_loop(0, 128, zero_wide, 0)

    def zero_narrow(i, carry):
      ob_v[i] = jnp.zeros((16,), jnp.float32)
      return carry

    lax.fori_loop(0, 128, zero_narrow, 0)
    pltpu.sync_copy(zb_v, pacc_sh.at[pl.ds(s * 128, 128)])
    pltpu.sync_copy(ob_v, cacc_sh.at[pl.ds(s * 128, 128)])

    def fill_ones(i, carry):
      ob_v[i] = jnp.ones((16,), jnp.float32)
      return carry

    lax.fori_loop(0, 128, fill_ones, 0)
    plsc.subcore_barrier()

    pltpu.sync_copy(bidx_hbm.at[pl.ds(w * PR_PW, PR_PW)], bi_v)
    for off, nb in ((0, 7), (7, 6)):
      pltpu.sync_copy(h2_hbm.at[pl.ds(w * PR_PW + off, nb)],
                      rows_v.at[pl.ds(0, nb)])
      pds = [pltpu.async_copy(rows_v.at[j], pacc_sh.at[bi_v.at[off + j]],
                              psem, add=True)
             for j in range(nb)]
      cds = [pltpu.async_copy(ob_v, cacc_sh.at[bi_v.at[off + j]], csem,
                              add=True)
             for j in range(nb)]
      for d in pds:
        d.wait()
      for d in cds:
        d.wait()
    plsc.subcore_barrier()
    pltpu.sync_copy(pacc_sh.at[pl.ds(s * 128, 128)],
                    pout_hbm.at[c].at[pl.ds(s * 128, 128)])
    pltpu.sync_copy(cacc_sh.at[pl.ds(s * 128, 128)],
                    cout_hbm.at[c].at[pl.ds(s * 128, 128)])

  return pool


_pool = _make_pool()


def _embed_body(x_ref, se_ref, ce_ref, pe_ref, o0_ref, o1_ref):
  xb = x_ref[...]
  oh0 = (xb[:, 0:1] ==
         lax.broadcasted_iota(jnp.int32, (RB, 8), 1)).astype(jnp.float32)
  oh1 = (xb[:, 1:2] ==
         lax.broadcasted_iota(jnp.int32, (RB, 8), 1)).astype(jnp.float32)
  p = jnp.clip(xb[:, 2:3], 0, 24)
  ohp = (p == lax.broadcasted_iota(jnp.int32, (RB, 32), 1)).astype(jnp.float32)
  h = (jnp.dot(oh0, se_ref[...], preferred_element_type=jnp.float32)
       + jnp.dot(oh1, ce_ref[...], preferred_element_type=jnp.float32)
       + jnp.dot(ohp, pe_ref[...], preferred_element_type=jnp.float32))
  o0_ref[...] = h[:, :16]
  o1_ref[...] = h[:, 16:]


def _embed(x, se, ce, pe_pad):
  full = lambda shp: pl.BlockSpec(shp, lambda n: tuple(0 for _ in shp))
  return pl.pallas_call(
      _embed_body,
      grid=(NGRID,),
      in_specs=[
          pl.BlockSpec((RB, 3), lambda n: (n, 0)),
          full((8, ED)), full((8, ED)), full((32, ED)),
      ],
      out_specs=[pl.BlockSpec((RB, 16), lambda n: (n, 0)),
                 pl.BlockSpec((RB, 16), lambda n: (n, 0))],
      out_shape=[jax.ShapeDtypeStruct((N, 16), jnp.float32),
                 jax.ShapeDtypeStruct((N, 16), jnp.float32)],
  )(x, se, ce, pe_pad)


def _dense_layer(h_chunks, s_raw_chunks, cnt_raw, W, root):
  C = len(h_chunks)
  din = 16 * C

  def body(*refs):
    h_refs = refs[:C]
    s_refs = refs[C:C + NREL * C]
    cnt_refs = refs[C + NREL * C:C + NREL * C + NREL]
    w_ref, root_ref = refs[C + NREL * C + NREL:C + NREL * C + NREL + 2]
    z_ref, st_ref = refs[-2:]

    n = pl.program_id(0)
    hb = jnp.concatenate([r[...] for r in h_refs], axis=1)
    z = jnp.dot(hb, root_ref[...], preferred_element_type=jnp.float32)
    for r in range(NREL):
      acc = jnp.zeros((RB, H), jnp.float32)
      for k in range(C):
        sk = s_refs[r * C + k][0]
        acc = acc + jnp.dot(sk, w_ref[r, 16 * k:16 * (k + 1), :],
                            preferred_element_type=jnp.float32)
      cntr = cnt_refs[r][0][:, 0]
      z = z + acc / jnp.maximum(cntr, 1.0)[:, None]
    z_ref[...] = z

    @pl.when(n == 0)
    def _():
      st_ref[...] = jnp.zeros((8, H), jnp.float32)

    st_ref[0:1, :] += jnp.sum(z, axis=0)[None]
    st_ref[1:2, :] += jnp.sum(z * z, axis=0)[None]

  def smap(r):
    return pl.BlockSpec(
        (1, RB, 16),
        lambda n, r=r: (n // (NH // RB), r * (NH // RB) + n % (NH // RB), 0))

  in_specs = ([pl.BlockSpec((RB, 16), lambda n: (n, 0))] * C
              + [smap(r) for r in range(NREL) for _ in range(C)]
              + [smap(r) for r in range(NREL)]
              + [pl.BlockSpec((NREL, din, H), lambda n: (0, 0, 0)),
                 pl.BlockSpec((din, H), lambda n: (0, 0))])
  operands = (list(h_chunks)
              + [s_raw_chunks[k] for _ in range(NREL) for k in range(C)]
              + [cnt_raw] * NREL + [W, root])
  return pl.pallas_call(
      body,
      grid=(NGRID,),
      in_specs=in_specs,
      out_specs=[pl.BlockSpec((RB, H), lambda n: (n, 0)),
                 pl.BlockSpec((8, H), lambda n: (0, 0))],
      out_shape=[jax.ShapeDtypeStruct((N, H), jnp.float32),
                 jax.ShapeDtypeStruct((8, H), jnp.float32)],
  )(*operands)


def _bn_relu(z, stats, gb, n_chunks):
  def body(z_ref, st_ref, gb_ref, *o_refs):
    zb = z_ref[...]
    mu = st_ref[0, :] / N
    var = st_ref[1, :] / N - mu * mu
    inv = lax.rsqrt(var + 1e-5)
    h = jnp.maximum(gb_ref[0, :] * (zb - mu) * inv + gb_ref[1, :], 0.0)
    if n_chunks == 0:
      o_refs[0][...] = h
    else:
      for k in range(n_chunks):
        o_refs[k][...] = h[:, 16 * k:16 * (k + 1)]

  if n_chunks == 0:
    out_specs = [pl.BlockSpec((RB, H), lambda n: (n, 0))]
    out_shape = [jax.ShapeDtypeStruct((N, H), jnp.float32)]
  else:
    out_specs = [pl.BlockSpec((RB, 16), lambda n: (n, 0))] * n_chunks
    out_shape = [jax.ShapeDtypeStruct((N, 16), jnp.float32)] * n_chunks
  res = pl.pallas_call(
      body,
      grid=(NGRID,),
      in_specs=[pl.BlockSpec((RB, H), lambda n: (n, 0)),
                pl.BlockSpec((8, H), lambda n: (0, 0)),
                pl.BlockSpec((8, H), lambda n: (0, 0))],
      out_specs=out_specs,
      out_shape=out_shape,
  )(z, stats, gb)
  return res


def _classifier(p0, p1, c0, c1, wpad, cbpad):
  def body(p0_ref, p1_ref, c0_ref, c1_ref, w_ref, cb_ref, o_ref):
    p = p0_ref[...] + p1_ref[...]
    cnt = c0_ref[...][:, 0:1] + c1_ref[...][:, 0:1]
    hg = p / jnp.maximum(cnt, 1.0)
    o_ref[...] = (jnp.dot(hg, w_ref[...], preferred_element_type=jnp.float32)
                  + cb_ref[0:1, :])

  return pl.pallas_call(
      body,
      out_shape=jax.ShapeDtypeStruct((GACC, 16), jnp.float32),
  )(p0, p1, c0, c1, wpad, cbpad)


def kernel(x, edge_index, edge_type, batch, shape_emb, color_emb, pos_emb,
           W1, root1, b1, g1, be1, W2, root2, b2, g2, be2, clsW, clsb):
  x = x.astype(jnp.int32)
  src = edge_index[0].astype(jnp.int32)
  dst = edge_index[1].astype(jnp.int32)
  et = edge_type.astype(jnp.int32)

  pad = EPAD - E
  src_p = jnp.concatenate([src, jnp.zeros((pad,), jnp.int32)])
  src_p = src_p.reshape(ER_TOT, 128)
  halves = []
  for c in range(NC):
    loc = et * NH + dst - c * NH
    ok = (dst >= c * NH) & (dst < (c + 1) * NH)
    hi = jnp.where(ok, loc, TRASH)
    halves.append(jnp.concatenate([hi, jnp.full((pad,), TRASH, jnp.int32)]))
  sidx2 = jnp.stack(halves).reshape(NC, ER_TOT, 128)
  zidx = jnp.zeros((ER_TOT, 128), jnp.int32)
  ones_tab = jnp.ones((8, 16), jnp.float32)

  pe_pad = jnp.zeros((32, ED), jnp.float32).at[:25].set(pos_emb)
  h0c = _embed(x, shape_emb, color_emb, pe_pad)

  cnt_raw = _edge_agg_cnt(ones_tab, zidx, sidx2)

  s1 = [_edge_agg_tab(h0c[k], src_p, sidx2) for k in range(ED // 16)]
  z1, st1 = _dense_layer(h0c, s1, cnt_raw, W1, root1)
  gb1 = jnp.zeros((8, H), jnp.float32).at[0].set(g1).at[1].set(be1)
  h1c = _bn_relu(z1, st1, gb1, H // 16)

  s2 = [_edge_agg_tab(h1c[k], src_p, sidx2) for k in range(H // 16)]
  z2, st2 = _dense_layer(h1c, s2, cnt_raw, W2, root2)
  gb2 = jnp.zeros((8, H), jnp.float32).at[0].set(g2).at[1].set(be2)
  h2 = _bn_relu(z2, st2, gb2, 0)[0]

  h2p = jnp.concatenate(
      [h2, jnp.zeros((NPOOL - N, H), jnp.float32)]).reshape(PR_TOT, 128, H)
  bidx = jnp.concatenate(
      [batch.astype(jnp.int32),
       jnp.full((NPOOL - N,), TRASHG, jnp.int32)]).reshape(PR_TOT, 128)
  pout, cout = _pool(h2p, bidx)

  wpad = jnp.zeros((H, 16), jnp.float32).at[:, :NCLS].set(clsW)
  cbpad = jnp.zeros((8, 16), jnp.float32).at[0, :NCLS].set(clsb)
  logits = _classifier(pout[0], pout[1], cout[0], cout[1], wpad, cbpad)
  return logits[:G, :NCLS]


# speedup vs baseline: 2.8876x; 1.0036x over previous
import functools

import jax
import jax.numpy as jnp
from jax import lax
from jax.experimental import pallas as pl
from jax.experimental.pallas import tpu as pltpu
from jax.experimental.pallas import tpu_sc as plsc

N = 50000
E = 800000
G = 2000
NREL = 3
ED = 32
H = 64
NCLS = 10

NC = 2
NS = 16
NH = N // NC
RACC = NREL * NH + 8
TRASH = NREL * NH
DRAIN = RACC // NS

NB = 16
OUT_IT = 25
ROWS_PW = OUT_IT * NB
ER_TOT = ROWS_PW * NS
EPAD = ER_TOT * 128

NPOOL = 53248
PR_TOT = NPOOL // 128
PR_PW = PR_TOT // (NC * NS)
GACC = 2048
TRASHG = 2040

RB = 200
NGRID = N // RB


def _make_edge_agg(tab_rows, gather=True):
  mesh = plsc.VectorSubcoreMesh(core_axis_name="c", subcore_axis_name="s")

  @functools.partial(
      pl.kernel,
      out_type=jax.ShapeDtypeStruct((NC, RACC, 16), jnp.float32),
      mesh=mesh,
      compiler_params=pltpu.CompilerParams(use_tc_tiling_on_sc=False),
      scratch_types=[
          pltpu.VMEM((NB * 128,), jnp.int32),
          pltpu.VMEM((NB * 128,), jnp.int32),
          pltpu.VMEM((NB * 128, 16), jnp.float32),
          pltpu.VMEM((128, 16), jnp.float32),
          pltpu.VMEM_SHARED((RACC, 16), jnp.float32),
          pltpu.SemaphoreType.DMA,
          pltpu.SemaphoreType.DMA,
      ],
  )
  def agg(tab_hbm, src_hbm, sidx_hbm, out_hbm,
          gi_v, si_v, rows_v, zb_v, acc_sh, gsem, ssem):
    c = lax.axis_index("c")
    s = lax.axis_index("s")

    def zero_body(i, carry):
      zb_v[i] = jnp.zeros((16,), jnp.float32)
      return carry

    lax.fori_loop(0, 128, zero_body, 0)

    def zero_copy(t, carry):
      pltpu.sync_copy(zb_v, acc_sh.at[pl.ds(s * DRAIN + t * 128, 128)])
      return carry

    lax.fori_loop(0, DRAIN // 128, zero_copy, 0)
    rem = DRAIN % 128
    if rem:
      pltpu.sync_copy(zb_v.at[pl.ds(0, rem)],
                      acc_sh.at[pl.ds(s * DRAIN + DRAIN - rem, rem)])
    if not gather:
      def ones_body(i, carry):
        rows_v[i] = jnp.ones((16,), jnp.float32)
        return carry

      lax.fori_loop(0, NB * 128, ones_body, 0)
    plsc.subcore_barrier()

    def outer(i, carry):
      eb = s * (ROWS_PW * 128) + i * (NB * 128)
      pltpu.sync_copy(sidx_hbm.at[c].at[pl.ds(eb, NB * 128)], si_v)
      if gather:
        pltpu.sync_copy(src_hbm.at[pl.ds(eb, NB * 128)], gi_v)
        pltpu.async_copy(tab_hbm.at[gi_v], rows_v, gsem).wait()
      pltpu.async_copy(rows_v, acc_sh.at[si_v], ssem, add=True).wait()
      return carry

    lax.fori_loop(0, OUT_IT, outer, 0)
    plsc.subcore_barrier()
    pltpu.sync_copy(acc_sh.at[pl.ds(s * DRAIN, DRAIN)],
                    out_hbm.at[c].at[pl.ds(s * DRAIN, DRAIN)])

  return agg


_edge_agg_tab = _make_edge_agg(N)
_edge_agg_cnt = _make_edge_agg(8, gather=False)


def _make_pool():
  mesh = plsc.VectorSubcoreMesh(core_axis_name="c", subcore_axis_name="s")

  @functools.partial(
      pl.kernel,
      out_type=(jax.ShapeDtypeStruct((NC, GACC, H), jnp.float32),
                jax.ShapeDtypeStruct((NC, GACC, 16), jnp.float32)),
      mesh=mesh,
      compiler_params=pltpu.CompilerParams(use_tc_tiling_on_sc=False),
      scratch_types=[
          pltpu.VMEM((PR_PW, 128), jnp.int32),
          pltpu.VMEM((7, 128, H), jnp.float32),
          pltpu.VMEM((128, H), jnp.float32),
          pltpu.VMEM((128, 16), jnp.float32),
          pltpu.VMEM_SHARED((GACC, H), jnp.float32),
          pltpu.VMEM_SHARED((GACC, 16), jnp.float32),
          pltpu.SemaphoreType.DMA,
          pltpu.SemaphoreType.DMA,
      ],
  )
  def pool(h2_hbm, bidx_hbm, pout_hbm, cout_hbm,
           bi_v, rows_v, zb_v, ob_v, pacc_sh, cacc_sh, psem, csem):
    c = lax.axis_index("c")
    s = lax.axis_index("s")
    w = s * NC + c

    def zero_wide(i, carry):
      for q in range(H // 16):
        zb_v[i, q * 16:(q + 1) * 16] = jnp.zeros((16,), jnp.float32)
      return carry

    lax.fori_loop(0, 128, zero_wide, 0)

    def zero_narrow(i, carry):
      ob_v[i] = jnp.zeros((16,), jnp.float32)
      return carry

    lax.fori_loop(0, 128, zero_narrow, 0)
    pltpu.sync_copy(zb_v, pacc_sh.at[pl.ds(s * 128, 128)])
    pltpu.sync_copy(ob_v, cacc_sh.at[pl.ds(s * 128, 128)])

    def fill_ones(i, carry):
      ob_v[i] = jnp.ones((16,), jnp.float32)
      return carry

    lax.fori_loop(0, 128, fill_ones, 0)
    plsc.subcore_barrier()

    pltpu.sync_copy(bidx_hbm.at[pl.ds(w * PR_PW, PR_PW)], bi_v)
    for off, nb in ((0, 7), (7, 6)):
      pltpu.sync_copy(h2_hbm.at[pl.ds(w * PR_PW + off, nb)],
                      rows_v.at[pl.ds(0, nb)])
      pds = [pltpu.async_copy(rows_v.at[j], pacc_sh.at[bi_v.at[off + j]],
                              psem, add=True)
             for j in range(nb)]
      cds = [pltpu.async_copy(ob_v, cacc_sh.at[bi_v.at[off + j]], csem,
                              add=True)
             for j in range(nb)]
      for d in pds:
        d.wait()
      for d in cds:
        d.wait()
    plsc.subcore_barrier()
    pltpu.sync_copy(pacc_sh.at[pl.ds(s * 128, 128)],
                    pout_hbm.at[c].at[pl.ds(s * 128, 128)])
    pltpu.sync_copy(cacc_sh.at[pl.ds(s * 128, 128)],
                    cout_hbm.at[c].at[pl.ds(s * 128, 128)])

  return pool


_pool = _make_pool()


def _embed_body(x_ref, se_ref, ce_ref, pe_ref, o0_ref, o1_ref):
  xb = x_ref[...]
  oh0 = (xb[:, 0:1] ==
         lax.broadcasted_iota(jnp.int32, (RB, 8), 1)).astype(jnp.float32)
  oh1 = (xb[:, 1:2] ==
         lax.broadcasted_iota(jnp.int32, (RB, 8), 1)).astype(jnp.float32)
  p = jnp.clip(xb[:, 2:3], 0, 24)
  ohp = (p == lax.broadcasted_iota(jnp.int32, (RB, 32), 1)).astype(jnp.float32)
  h = (jnp.dot(oh0, se_ref[...], preferred_element_type=jnp.float32)
       + jnp.dot(oh1, ce_ref[...], preferred_element_type=jnp.float32)
       + jnp.dot(ohp, pe_ref[...], preferred_element_type=jnp.float32))
  o0_ref[...] = h[:, :16]
  o1_ref[...] = h[:, 16:]


def _embed(x, se, ce, pe_pad):
  full = lambda shp: pl.BlockSpec(shp, lambda n: tuple(0 for _ in shp))
  return pl.pallas_call(
      _embed_body,
      grid=(NGRID,),
      in_specs=[
          pl.BlockSpec((RB, 3), lambda n: (n, 0)),
          full((8, ED)), full((8, ED)), full((32, ED)),
      ],
      out_specs=[pl.BlockSpec((RB, 16), lambda n: (n, 0)),
                 pl.BlockSpec((RB, 16), lambda n: (n, 0))],
      out_shape=[jax.ShapeDtypeStruct((N, 16), jnp.float32),
                 jax.ShapeDtypeStruct((N, 16), jnp.float32)],
  )(x, se, ce, pe_pad)


def _dense_layer(h_chunks, s_raw_chunks, cnt_raw, W, root):
  C = len(h_chunks)
  din = 16 * C

  def body(*refs):
    h_refs = refs[:C]
    s_refs = refs[C:C + NREL * C]
    cnt_refs = refs[C + NREL * C:C + NREL * C + NREL]
    w_ref, root_ref = refs[C + NREL * C + NREL:C + NREL * C + NREL + 2]
    z_ref, st_ref = refs[-2:]

    n = pl.program_id(0)
    hb = jnp.concatenate([r[...] for r in h_refs], axis=1)
    z = jnp.dot(hb, root_ref[...], preferred_element_type=jnp.float32)
    for r in range(NREL):
      acc = jnp.zeros((RB, H), jnp.float32)
      for k in range(C):
        sk = s_refs[r * C + k][0]
        acc = acc + jnp.dot(sk, w_ref[r, 16 * k:16 * (k + 1), :],
                            preferred_element_type=jnp.float32)
      cntr = cnt_refs[r][0][:, 0]
      z = z + acc / jnp.maximum(cntr, 1.0)[:, None]
    z_ref[...] = z

    @pl.when(n == 0)
    def _():
      st_ref[...] = jnp.zeros((8, H), jnp.float32)

    st_ref[0:1, :] += jnp.sum(z, axis=0)[None]
    st_ref[1:2, :] += jnp.sum(z * z, axis=0)[None]

  def smap(r):
    return pl.BlockSpec(
        (1, RB, 16),
        lambda n, r=r: (n // (NH // RB), r * (NH // RB) + n % (NH // RB), 0))

  in_specs = ([pl.BlockSpec((RB, 16), lambda n: (n, 0))] * C
              + [smap(r) for r in range(NREL) for _ in range(C)]
              + [smap(r) for r in range(NREL)]
              + [pl.BlockSpec((NREL, din, H), lambda n: (0, 0, 0)),
                 pl.BlockSpec((din, H), lambda n: (0, 0))])
  operands = (list(h_chunks)
              + [s_raw_chunks[k] for _ in range(NREL) for k in range(C)]
              + [cnt_raw] * NREL + [W, root])
  return pl.pallas_call(
      body,
      grid=(NGRID,),
      in_specs=in_specs,
      out_specs=[pl.BlockSpec((RB, H), lambda n: (n, 0)),
                 pl.BlockSpec((8, H), lambda n: (0, 0))],
      out_shape=[jax.ShapeDtypeStruct((N, H), jnp.float32),
                 jax.ShapeDtypeStruct((8, H), jnp.float32)],
  )(*operands)


def _bn_relu(z, stats, gb, n_chunks):
  def body(z_ref, st_ref, gb_ref, *o_refs):
    zb = z_ref[...]
    mu = st_ref[0, :] / N
    var = st_ref[1, :] / N - mu * mu
    inv = lax.rsqrt(var + 1e-5)
    h = jnp.maximum(gb_ref[0, :] * (zb - mu) * inv + gb_ref[1, :], 0.0)
    if n_chunks == 0:
      o_refs[0][...] = h
    else:
      for k in range(n_chunks):
        o_refs[k][...] = h[:, 16 * k:16 * (k + 1)]

  if n_chunks == 0:
    out_specs = [pl.BlockSpec((RB, H), lambda n: (n, 0))]
    out_shape = [jax.ShapeDtypeStruct((N, H), jnp.float32)]
  else:
    out_specs = [pl.BlockSpec((RB, 16), lambda n: (n, 0))] * n_chunks
    out_shape = [jax.ShapeDtypeStruct((N, 16), jnp.float32)] * n_chunks
  res = pl.pallas_call(
      body,
      grid=(NGRID,),
      in_specs=[pl.BlockSpec((RB, H), lambda n: (n, 0)),
                pl.BlockSpec((8, H), lambda n: (0, 0)),
                pl.BlockSpec((8, H), lambda n: (0, 0))],
      out_specs=out_specs,
      out_shape=out_shape,
  )(z, stats, gb)
  return res


def _classifier(p0, p1, c0, c1, wpad, cbpad):
  def body(p0_ref, p1_ref, c0_ref, c1_ref, w_ref, cb_ref, o_ref):
    p = p0_ref[...] + p1_ref[...]
    cnt = c0_ref[...][:, 0:1] + c1_ref[...][:, 0:1]
    hg = p / jnp.maximum(cnt, 1.0)
    o_ref[...] = (jnp.dot(hg, w_ref[...], preferred_element_type=jnp.float32)
                  + cb_ref[0:1, :])

  return pl.pallas_call(
      body,
      out_shape=jax.ShapeDtypeStruct((GACC, 16), jnp.float32),
  )(p0, p1, c0, c1, wpad, cbpad)


def kernel(x, edge_index, edge_type, batch, shape_emb, color_emb, pos_emb,
           W1, root1, b1, g1, be1, W2, root2, b2, g2, be2, clsW, clsb):
  x = x.astype(jnp.int32)
  src = edge_index[0].astype(jnp.int32)
  dst = edge_index[1].astype(jnp.int32)
  et = edge_type.astype(jnp.int32)

  pad = EPAD - E
  src_p = jnp.concatenate([src, jnp.zeros((pad,), jnp.int32)])
  halves = []
  for c in range(NC):
    loc = et * NH + dst - c * NH
    ok = (dst >= c * NH) & (dst < (c + 1) * NH)
    hi = jnp.where(ok, loc, TRASH)
    halves.append(jnp.concatenate([hi, jnp.full((pad,), TRASH, jnp.int32)]))
  sidx2 = jnp.stack(halves)
  zidx = jnp.zeros((EPAD,), jnp.int32)
  ones_tab = jnp.ones((8, 16), jnp.float32)

  pe_pad = jnp.zeros((32, ED), jnp.float32).at[:25].set(pos_emb)
  h0c = _embed(x, shape_emb, color_emb, pe_pad)

  cnt_raw = _edge_agg_cnt(ones_tab, zidx, sidx2)

  s1 = [_edge_agg_tab(h0c[k], src_p, sidx2) for k in range(ED // 16)]
  z1, st1 = _dense_layer(h0c, s1, cnt_raw, W1, root1)
  gb1 = jnp.zeros((8, H), jnp.float32).at[0].set(g1).at[1].set(be1)
  h1c = _bn_relu(z1, st1, gb1, H // 16)

  s2 = [_edge_agg_tab(h1c[k], src_p, sidx2) for k in range(H // 16)]
  z2, st2 = _dense_layer(h1c, s2, cnt_raw, W2, root2)
  gb2 = jnp.zeros((8, H), jnp.float32).at[0].set(g2).at[1].set(be2)
  h2 = _bn_relu(z2, st2, gb2, 0)[0]

  h2p = jnp.concatenate(
      [h2, jnp.zeros((NPOOL - N, H), jnp.float32)]).reshape(PR_TOT, 128, H)
  bidx = jnp.concatenate(
      [batch.astype(jnp.int32),
       jnp.full((NPOOL - N,), TRASHG, jnp.int32)]).reshape(PR_TOT, 128)
  pout, cout = _pool(h2p, bidx)

  wpad = jnp.zeros((H, 16), jnp.float32).at[:, :NCLS].set(clsW)
  cbpad = jnp.zeros((8, 16), jnp.float32).at[0, :NCLS].set(clsb)
  logits = _classifier(pout[0], pout[1], cout[0], cout[1], wpad, cbpad)
  return logits[:G, :NCLS]
